# baseline (device time: 108221 ns/iter reference)
import jax
import jax.numpy as jnp
from jax import lax
from jax.experimental import pallas as pl
from jax.experimental.pallas import tpu as pltpu

N_DEV = 8
M = 2048
D = 2048
H = M // 2
R = H // N_DEV
NSUB = 4
SUB = R // NSUB
NH = N_DEV - 1


def _body(
    x_ref,
    resid_ref,
    gamma_ref,
    out_ref,
    xb_ref,
    rs_send_r,
    rs_recv_r,
    rs_send_l,
    rs_recv_l,
    rs_send_sem_r,
    rs_recv_sem_r,
    rs_send_sem_l,
    rs_recv_sem_l,
    ag_send_sem_r,
    ag_recv_sem_r,
    ag_send_sem_l,
    ag_recv_sem_l,
):
    i = lax.axis_index("i")
    right = lax.rem(i + 1, N_DEV)
    left = lax.rem(i + N_DEV - 1, N_DEV)

    dirs = {
        "r": dict(
            dev=right,
            base=0,
            rs_send=rs_send_r,
            rs_recv=rs_recv_r,
            rs_ssem=rs_send_sem_r,
            rs_rsem=rs_recv_sem_r,
            ag_ssem=ag_send_sem_r,
            ag_rsem=ag_recv_sem_r,
            rs_chunk=lambda s: lax.rem(i - s + N_DEV, N_DEV),
            ag_chunk=lambda s: lax.rem(i + 1 - s + N_DEV, N_DEV),
        ),
        "l": dict(
            dev=left,
            base=H,
            rs_send=rs_send_l,
            rs_recv=rs_recv_l,
            rs_ssem=rs_send_sem_l,
            rs_rsem=rs_recv_sem_l,
            ag_ssem=ag_send_sem_l,
            ag_rsem=ag_recv_sem_l,
            rs_chunk=lambda s: lax.rem(i + s, N_DEV),
            ag_chunk=lambda s: lax.rem(i + N_DEV - 1 + s, N_DEV),
        ),
    }

    def sub_rows(base, c, j):
        return pl.ds(base + c * R + j * SUB, SUB)

    xb_ref[:, :] = x_ref[:, :].astype(jnp.bfloat16)

    barrier_sem = pltpu.get_barrier_semaphore()
    for nbr in (left, right):
        pl.semaphore_signal(
            barrier_sem,
            inc=1,
            device_id=(nbr,),
            device_id_type=pl.DeviceIdType.MESH,
        )
    pl.semaphore_wait(barrier_sem, 2)

    rs_desc = {}

    def rs_start(s, d, j):
        dd = dirs[d]
        desc = pltpu.make_async_remote_copy(
            src_ref=dd["rs_send"].at[s % 2, pl.ds(j * SUB, SUB)],
            dst_ref=dd["rs_recv"].at[s, pl.ds(j * SUB, SUB)],
            send_sem=dd["rs_ssem"].at[s % 2, j],
            recv_sem=dd["rs_rsem"].at[s, j],
            device_id=(dd["dev"],),
            device_id_type=pl.DeviceIdType.MESH,
        )
        rs_desc[(s, d, j)] = desc
        desc.start()

    for d in ("r", "l"):
        dd = dirs[d]
        dd["rs_send"][0, :, :] = xb_ref[pl.ds(dd["base"] + i * R, R), :]
    for j in range(NSUB):
        for d in ("r", "l"):
            rs_start(0, d, j)

    for s in range(1, NH):
        for j in range(NSUB):
            for d in ("r", "l"):
                dd = dirs[d]
                rs_desc[(s - 1, d, j)].wait_recv()
                if s >= 2:
                    rs_desc[(s - 2, d, j)].wait_send()
                c = dd["rs_chunk"](s)
                dd["rs_send"][s % 2, pl.ds(j * SUB, SUB), :] = (
                    dd["rs_recv"][s - 1, pl.ds(j * SUB, SUB), :]
                    + xb_ref[sub_rows(dd["base"], c, j), :]
                )
                rs_start(s, d, j)

    g = gamma_ref[:, :]
    ag_desc = {}

    def ag_start(s, d, j):
        dd = dirs[d]
        rows = sub_rows(dd["base"], dd["ag_chunk"](s), j)
        desc = pltpu.make_async_remote_copy(
            src_ref=out_ref.at[rows],
            dst_ref=out_ref.at[rows],
            send_sem=dd["ag_ssem"].at[s % 2, j],
            recv_sem=dd["ag_rsem"].at[s, j],
            device_id=(dd["dev"],),
            device_id_type=pl.DeviceIdType.MESH,
        )
        ag_desc[(s, d, j)] = desc
        desc.start()

    for j in range(NSUB):
        for d in ("r", "l"):
            dd = dirs[d]
            rs_desc[(NH - 1, d, j)].wait_recv()
            c = dd["ag_chunk"](0)
            rows = sub_rows(dd["base"], c, j)
            y = (
                dd["rs_recv"][NH - 1, pl.ds(j * SUB, SUB), :].astype(jnp.float32)
                + x_ref[rows, :]
                + resid_ref[rows, :]
            )
            rms = jnp.sqrt(jnp.mean(y * y, axis=-1, keepdims=True) + 1e-6)
            out_ref[rows, :] = ((y / rms) * g).astype(jnp.bfloat16)
            ag_start(0, d, j)

    for s in range(1, NH):
        for j in range(NSUB):
            for d in ("r", "l"):
                ag_desc[(s - 1, d, j)].wait_recv()
                if s >= 2:
                    ag_desc[(s - 2, d, j)].wait_send()
                ag_start(s, d, j)

    for j in range(NSUB):
        for d in ("r", "l"):
            ag_desc[(NH - 1, d, j)].wait_recv()
            rs_desc[(NH - 2, d, j)].wait_send()
            rs_desc[(NH - 1, d, j)].wait_send()
            ag_desc[(NH - 2, d, j)].wait_send()
            ag_desc[(NH - 1, d, j)].wait_send()


def kernel(partial, resid, gamma):
    x = partial.reshape(M, D)
    g = gamma.reshape(1, D)
    return pl.pallas_call(
        _body,
        out_shape=jax.ShapeDtypeStruct((M, D), jnp.bfloat16),
        in_specs=[
            pl.BlockSpec(memory_space=pltpu.VMEM),
            pl.BlockSpec(memory_space=pltpu.VMEM),
            pl.BlockSpec(memory_space=pltpu.VMEM),
        ],
        out_specs=pl.BlockSpec(memory_space=pltpu.VMEM),
        scratch_shapes=[
            pltpu.VMEM((M, D), jnp.bfloat16),
            pltpu.VMEM((2, R, D), jnp.bfloat16),
            pltpu.VMEM((NH, R, D), jnp.bfloat16),
            pltpu.VMEM((2, R, D), jnp.bfloat16),
            pltpu.VMEM((NH, R, D), jnp.bfloat16),
            pltpu.SemaphoreType.DMA((2, NSUB)),
            pltpu.SemaphoreType.DMA((NH, NSUB)),
            pltpu.SemaphoreType.DMA((2, NSUB)),
            pltpu.SemaphoreType.DMA((NH, NSUB)),
            pltpu.SemaphoreType.DMA((2, NSUB)),
            pltpu.SemaphoreType.DMA((NH, NSUB)),
            pltpu.SemaphoreType.DMA((2, NSUB)),
            pltpu.SemaphoreType.DMA((NH, NSUB)),
        ],
        compiler_params=pltpu.CompilerParams(
            collective_id=0, vmem_limit_bytes=96 * 1024 * 1024
        ),
    )(x, resid, g)


# device time: 102785 ns/iter; 1.0529x vs baseline; 1.0529x over previous
import jax
import jax.numpy as jnp
from jax import lax
from jax.experimental import pallas as pl
from jax.experimental.pallas import tpu as pltpu

N_DEV = 8
M = 2048
D = 2048
NB = 4
BROWS = M // NB
SEG = BROWS // N_DEV

MASKS = (
    (1, 3, 4),
    (3, 4, 1),
    (4, 3, 1),
    (1, 4, 3),
)

ORD_X_FIRST = (0, 3, 1, 2)
ORD_YZ_FIRST = (1, 2, 0, 3)


def _body(
    x_ref,
    resid_ref,
    gamma_ref,
    out_ref,
    xb_ref,
    rs1s,
    rs1r,
    rs2s,
    rs2r,
    rs3s,
    rs3r,
    rs_ssem,
    rs_rsem,
    ag_ssem,
    ag_rsem,
):
    i = lax.axis_index("i")

    xb_ref[:, :] = x_ref[:, :].astype(jnp.bfloat16)

    barrier_sem = pltpu.get_barrier_semaphore()
    for m in (1, 3, 4):
        pl.semaphore_signal(
            barrier_sem,
            inc=1,
            device_id=(i ^ m,),
            device_id_type=pl.DeviceIdType.MESH,
        )
    pl.semaphore_wait(barrier_sem, 3)

    def seg(b, o):
        return pl.ds(b * BROWS + o * SEG, SEG)

    rs_desc = {}

    def rs_start(rnd, b, send_buf, recv_buf, partner):
        desc = pltpu.make_async_remote_copy(
            src_ref=send_buf.at[b],
            dst_ref=recv_buf.at[b],
            send_sem=rs_ssem.at[b, rnd],
            recv_sem=rs_rsem.at[b, rnd],
            device_id=(partner,),
            device_id_type=pl.DeviceIdType.MESH,
        )
        rs_desc[(rnd, b)] = desc
        desc.start()

    for b in ORD_X_FIRST:
        m1, m2, m3 = MASKS[b]
        p1 = i ^ m1
        for q, e in enumerate((0, m2, m3, m2 ^ m3)):
            rs1s[b, pl.ds(q * SEG, SEG), :] = xb_ref[seg(b, p1 ^ e), :]
        rs_start(0, b, rs1s, rs1r, p1)

    for b in ORD_YZ_FIRST:
        m1, m2, m3 = MASKS[b]
        p2 = i ^ m2
        rs_desc[(0, b)].wait_recv()
        for q2, (e, q1) in enumerate(((0, 1), (m3, 3))):
            rs2s[b, pl.ds(q2 * SEG, SEG), :] = (
                xb_ref[seg(b, p2 ^ e), :] + rs1r[b, pl.ds(q1 * SEG, SEG), :]
            )
        rs_start(1, b, rs2s, rs2r, p2)

    for b in ORD_YZ_FIRST:
        m1, m2, m3 = MASKS[b]
        p3 = i ^ m3
        rs_desc[(1, b)].wait_recv()
        rs3s[b, :, :] = (
            xb_ref[seg(b, p3), :]
            + rs1r[b, pl.ds(2 * SEG, SEG), :]
            + rs2r[b, pl.ds(1 * SEG, SEG), :]
        )
        rs_start(2, b, rs3s, rs3r, p3)

    g = gamma_ref[:, :]
    ag_desc = {}
    AG_IDX = {0: (0,), 1: (1, 2), 2: (3, 4, 5, 6)}

    def ag_start(rnd, b, owners, partner):
        for k, o in enumerate(owners):
            idx = AG_IDX[rnd][k]
            desc = pltpu.make_async_remote_copy(
                src_ref=out_ref.at[seg(b, o)],
                dst_ref=out_ref.at[seg(b, o)],
                send_sem=ag_ssem.at[b, idx],
                recv_sem=ag_rsem.at[b, idx],
                device_id=(partner,),
                device_id_type=pl.DeviceIdType.MESH,
            )
            ag_desc[(rnd, b, k)] = desc
            desc.start()

    for b in ORD_YZ_FIRST:
        m1, m2, m3 = MASKS[b]
        rs_desc[(2, b)].wait_recv()
        rows = seg(b, i)
        y = (
            x_ref[rows, :]
            + resid_ref[rows, :]
            + rs1r[b, pl.ds(0, SEG), :].astype(jnp.float32)
            + rs2r[b, pl.ds(0, SEG), :].astype(jnp.float32)
            + rs3r[b, :, :].astype(jnp.float32)
        )
        rms = jnp.sqrt(jnp.mean(y * y, axis=-1, keepdims=True) + 1e-6)
        out_ref[rows, :] = ((y / rms) * g).astype(jnp.bfloat16)
        ag_start(0, b, (i,), i ^ m3)

    for b in ORD_YZ_FIRST:
        m1, m2, m3 = MASKS[b]
        ag_desc[(0, b, 0)].wait_recv()
        ag_start(1, b, (i, i ^ m3), i ^ m2)

    for b in ORD_X_FIRST:
        m1, m2, m3 = MASKS[b]
        ag_desc[(1, b, 0)].wait_recv()
        ag_desc[(1, b, 1)].wait_recv()
        ag_start(2, b, (i, i ^ m3, i ^ m2, i ^ m2 ^ m3), i ^ m1)

    for b in ORD_YZ_FIRST:
        for k in range(4):
            ag_desc[(2, b, k)].wait_recv()
    for b in range(NB):
        for rnd in range(3):
            rs_desc[(rnd, b)].wait_send()
        ag_desc[(0, b, 0)].wait_send()
        for k in range(2):
            ag_desc[(1, b, k)].wait_send()
        for k in range(4):
            ag_desc[(2, b, k)].wait_send()


def kernel(partial, resid, gamma):
    x = partial.reshape(M, D)
    g = gamma.reshape(1, D)
    return pl.pallas_call(
        _body,
        out_shape=jax.ShapeDtypeStruct((M, D), jnp.bfloat16),
        in_specs=[
            pl.BlockSpec(memory_space=pltpu.VMEM),
            pl.BlockSpec(memory_space=pltpu.VMEM),
            pl.BlockSpec(memory_space=pltpu.VMEM),
        ],
        out_specs=pl.BlockSpec(memory_space=pltpu.VMEM),
        scratch_shapes=[
            pltpu.VMEM((M, D), jnp.bfloat16),
            pltpu.VMEM((NB, 4 * SEG, D), jnp.bfloat16),
            pltpu.VMEM((NB, 4 * SEG, D), jnp.bfloat16),
            pltpu.VMEM((NB, 2 * SEG, D), jnp.bfloat16),
            pltpu.VMEM((NB, 2 * SEG, D), jnp.bfloat16),
            pltpu.VMEM((NB, SEG, D), jnp.bfloat16),
            pltpu.VMEM((NB, SEG, D), jnp.bfloat16),
            pltpu.SemaphoreType.DMA((NB, 3)),
            pltpu.SemaphoreType.DMA((NB, 3)),
            pltpu.SemaphoreType.DMA((NB, 7)),
            pltpu.SemaphoreType.DMA((NB, 7)),
        ],
        compiler_params=pltpu.CompilerParams(
            collective_id=0, vmem_limit_bytes=96 * 1024 * 1024
        ),
    )(x, resid, g)


# device time: 85404 ns/iter; 1.2672x vs baseline; 1.2035x over previous
import jax
import jax.numpy as jnp
from jax import lax
from jax.experimental import pallas as pl
from jax.experimental.pallas import tpu as pltpu

N_DEV = 8
M = 2048
D = 2048
NB = 4
BROWS = M // NB
SEG = BROWS // N_DEV

MASKS = (
    (1, 3, 4),
    (3, 4, 1),
    (4, 3, 1),
    (1, 4, 3),
)

ORD_X_FIRST = (0, 3, 1, 2)
ORD_YZ_FIRST = (1, 2, 0, 3)


def _body(
    x_ref,
    resid_ref,
    gamma_ref,
    out_ref,
    xb_ref,
    rs1r,
    rs2s,
    rs2r,
    rs3s,
    rs3r,
    rs1_ssem,
    rs1_rsem,
    rs2_ssem,
    rs2_rsem,
    rs3_ssem,
    rs3_rsem,
    ag_ssem,
    ag_rsem,
):
    i = lax.axis_index("i")

    xb_ref[:, :] = x_ref[:, :].astype(jnp.bfloat16)

    barrier_sem = pltpu.get_barrier_semaphore()
    for m in (1, 3, 4):
        pl.semaphore_signal(
            barrier_sem,
            inc=1,
            device_id=(i ^ m,),
            device_id_type=pl.DeviceIdType.MESH,
        )
    pl.semaphore_wait(barrier_sem, 3)

    def seg(b, o):
        return pl.ds(b * BROWS + o * SEG, SEG)

    def srows(q):
        return pl.ds(q * SEG, SEG)

    desc = {}

    def start(key, src, dst, ssem, rsem, partner):
        d = pltpu.make_async_remote_copy(
            src_ref=src,
            dst_ref=dst,
            send_sem=ssem,
            recv_sem=rsem,
            device_id=(partner,),
            device_id_type=pl.DeviceIdType.MESH,
        )
        desc[key] = d
        d.start()

    for b in ORD_X_FIRST:
        m1, m2, m3 = MASKS[b]
        p1 = i ^ m1
        for q, e in ((1, m2), (3, m2 ^ m3), (0, 0), (2, m3)):
            start(
                ("rs1", b, q),
                xb_ref.at[seg(b, p1 ^ e)],
                rs1r.at[b, srows(q)],
                rs1_ssem.at[b, q],
                rs1_rsem.at[b, q],
                p1,
            )

    for b in ORD_YZ_FIRST:
        m1, m2, m3 = MASKS[b]
        p2 = i ^ m2
        desc[("rs1", b, 1)].wait_recv()
        desc[("rs1", b, 3)].wait_recv()
        rs2s[b, srows(1), :] = xb_ref[seg(b, p2 ^ m3), :] + rs1r[b, srows(3), :]
        start(
            ("rs2", b, 1),
            rs2s.at[b, srows(1)],
            rs2r.at[b, srows(1)],
            rs2_ssem.at[b, 1],
            rs2_rsem.at[b, 1],
            p2,
        )
        rs2s[b, srows(0), :] = xb_ref[seg(b, p2), :] + rs1r[b, srows(1), :]
        start(
            ("rs2", b, 0),
            rs2s.at[b, srows(0)],
            rs2r.at[b, srows(0)],
            rs2_ssem.at[b, 0],
            rs2_rsem.at[b, 0],
            p2,
        )

    for b in ORD_YZ_FIRST:
        m1, m2, m3 = MASKS[b]
        p3 = i ^ m3
        desc[("rs2", b, 1)].wait_recv()
        desc[("rs1", b, 2)].wait_recv()
        rs3s[b, :, :] = (
            xb_ref[seg(b, p3), :]
            + rs1r[b, srows(2), :]
            + rs2r[b, srows(1), :]
        )
        start(
            ("rs3", b),
            rs3s.at[b],
            rs3r.at[b],
            rs3_ssem.at[b],
            rs3_rsem.at[b],
            p3,
        )

    g = gamma_ref[:, :]

    def ag(idx, b, o, partner):
        start(
            ("ag", b, idx),
            out_ref.at[seg(b, o)],
            out_ref.at[seg(b, o)],
            ag_ssem.at[b, idx],
            ag_rsem.at[b, idx],
            partner,
        )

    for b in ORD_X_FIRST:
        m1, m2, m3 = MASKS[b]
        desc[("rs3", b)].wait_recv()
        desc[("rs1", b, 0)].wait_recv()
        desc[("rs2", b, 0)].wait_recv()
        rows = seg(b, i)
        y = (
            x_ref[rows, :]
            + resid_ref[rows, :]
            + rs1r[b, srows(0), :].astype(jnp.float32)
            + rs2r[b, srows(0), :].astype(jnp.float32)
            + rs3r[b, :, :].astype(jnp.float32)
        )
        rms = jnp.sqrt(jnp.mean(y * y, axis=-1, keepdims=True) + 1e-6)
        out_ref[rows, :] = ((y / rms) * g).astype(jnp.bfloat16)
        ag(0, b, i, i ^ m3)
        ag(1, b, i, i ^ m2)
        ag(3, b, i, i ^ m1)

    for b in ORD_X_FIRST:
        m1, m2, m3 = MASKS[b]
        desc[("ag", b, 0)].wait_recv()
        ag(2, b, i ^ m3, i ^ m2)
        ag(4, b, i ^ m3, i ^ m1)

    for b in ORD_X_FIRST:
        m1, m2, m3 = MASKS[b]
        desc[("ag", b, 1)].wait_recv()
        desc[("ag", b, 2)].wait_recv()
        ag(5, b, i ^ m2, i ^ m1)
        ag(6, b, i ^ m2 ^ m3, i ^ m1)

    for b in ORD_YZ_FIRST:
        for idx in (3, 4, 5, 6):
            desc[("ag", b, idx)].wait_recv()
    for b in range(NB):
        for q in range(4):
            desc[("rs1", b, q)].wait_send()
        for q in range(2):
            desc[("rs2", b, q)].wait_send()
        desc[("rs3", b)].wait_send()
        for idx in range(7):
            desc[("ag", b, idx)].wait_send()


def kernel(partial, resid, gamma):
    x = partial.reshape(M, D)
    g = gamma.reshape(1, D)
    return pl.pallas_call(
        _body,
        out_shape=jax.ShapeDtypeStruct((M, D), jnp.bfloat16),
        in_specs=[
            pl.BlockSpec(memory_space=pltpu.VMEM),
            pl.BlockSpec(memory_space=pltpu.VMEM),
            pl.BlockSpec(memory_space=pltpu.VMEM),
        ],
        out_specs=pl.BlockSpec(memory_space=pltpu.VMEM),
        scratch_shapes=[
            pltpu.VMEM((M, D), jnp.bfloat16),
            pltpu.VMEM((NB, 4 * SEG, D), jnp.bfloat16),
            pltpu.VMEM((NB, 2 * SEG, D), jnp.bfloat16),
            pltpu.VMEM((NB, 2 * SEG, D), jnp.bfloat16),
            pltpu.VMEM((NB, SEG, D), jnp.bfloat16),
            pltpu.VMEM((NB, SEG, D), jnp.bfloat16),
            pltpu.SemaphoreType.DMA((NB, 4)),
            pltpu.SemaphoreType.DMA((NB, 4)),
            pltpu.SemaphoreType.DMA((NB, 2)),
            pltpu.SemaphoreType.DMA((NB, 2)),
            pltpu.SemaphoreType.DMA((NB,)),
            pltpu.SemaphoreType.DMA((NB,)),
            pltpu.SemaphoreType.DMA((NB, 7)),
            pltpu.SemaphoreType.DMA((NB, 7)),
        ],
        compiler_params=pltpu.CompilerParams(
            collective_id=0, vmem_limit_bytes=96 * 1024 * 1024
        ),
    )(x, resid, g)


# device time: 85378 ns/iter; 1.2676x vs baseline; 1.0003x over previous
import jax
import jax.numpy as jnp
from jax import lax
from jax.experimental import pallas as pl
from jax.experimental.pallas import tpu as pltpu

N_DEV = 8
M = 2048
D = 2048
NB = 4
BROWS = M // NB
SEG = BROWS // N_DEV

MASKS = (
    (1, 3, 4),
    (3, 4, 1),
    (4, 3, 1),
    (1, 4, 3),
)

ORD_X_FIRST = (0, 3, 1, 2)
ORD_YZ_FIRST = (1, 2, 0, 3)


def _body(
    x_ref,
    resid_ref,
    gamma_ref,
    out_ref,
    xb_ref,
    rs1r,
    rs2s,
    rs2r,
    rs3s,
    rs3r,
    rs1_ssem,
    rs1_rsem,
    rs2_ssem,
    rs2_rsem,
    rs3_ssem,
    rs3_rsem,
    ag_ssem,
    ag_rsem,
):
    i = lax.axis_index("i")

    xb_ref[:, :] = x_ref[0, :, :].astype(jnp.bfloat16)

    barrier_sem = pltpu.get_barrier_semaphore()
    for m in (1, 3, 4):
        pl.semaphore_signal(
            barrier_sem,
            inc=1,
            device_id=(i ^ m,),
            device_id_type=pl.DeviceIdType.MESH,
        )
    pl.semaphore_wait(barrier_sem, 3)

    def seg(b, o):
        return pl.ds(b * BROWS + o * SEG, SEG)

    def srows(q):
        return pl.ds(q * SEG, SEG)

    desc = {}

    def start(key, src, dst, ssem, rsem, partner):
        d = pltpu.make_async_remote_copy(
            src_ref=src,
            dst_ref=dst,
            send_sem=ssem,
            recv_sem=rsem,
            device_id=(partner,),
            device_id_type=pl.DeviceIdType.MESH,
        )
        desc[key] = d
        d.start()

    for b in ORD_X_FIRST:
        m1, m2, m3 = MASKS[b]
        p1 = i ^ m1
        for q, e in ((1, m2), (3, m2 ^ m3), (0, 0), (2, m3)):
            start(
                ("rs1", b, q),
                xb_ref.at[seg(b, p1 ^ e)],
                rs1r.at[b, srows(q)],
                rs1_ssem.at[b, q],
                rs1_rsem.at[b, q],
                p1,
            )

    for b in ORD_YZ_FIRST:
        m1, m2, m3 = MASKS[b]
        p2 = i ^ m2
        desc[("rs1", b, 1)].wait_recv()
        desc[("rs1", b, 3)].wait_recv()
        rs2s[b, srows(1), :] = xb_ref[seg(b, p2 ^ m3), :] + rs1r[b, srows(3), :]
        start(
            ("rs2", b, 1),
            rs2s.at[b, srows(1)],
            rs2r.at[b, srows(1)],
            rs2_ssem.at[b, 1],
            rs2_rsem.at[b, 1],
            p2,
        )
        rs2s[b, srows(0), :] = xb_ref[seg(b, p2), :] + rs1r[b, srows(1), :]
        start(
            ("rs2", b, 0),
            rs2s.at[b, srows(0)],
            rs2r.at[b, srows(0)],
            rs2_ssem.at[b, 0],
            rs2_rsem.at[b, 0],
            p2,
        )

    for b in ORD_YZ_FIRST:
        m1, m2, m3 = MASKS[b]
        p3 = i ^ m3
        desc[("rs2", b, 1)].wait_recv()
        desc[("rs1", b, 2)].wait_recv()
        rs3s[b, :, :] = (
            xb_ref[seg(b, p3), :]
            + rs1r[b, srows(2), :]
            + rs2r[b, srows(1), :]
        )
        start(
            ("rs3", b),
            rs3s.at[b],
            rs3r.at[b],
            rs3_ssem.at[b],
            rs3_rsem.at[b],
            p3,
        )

    g = gamma_ref[:, :]

    def ag(idx, b, o, partner):
        start(
            ("ag", b, idx),
            out_ref.at[seg(b, o)],
            out_ref.at[seg(b, o)],
            ag_ssem.at[b, idx],
            ag_rsem.at[b, idx],
            partner,
        )

    for b in ORD_X_FIRST:
        m1, m2, m3 = MASKS[b]
        desc[("rs3", b)].wait_recv()
        desc[("rs1", b, 0)].wait_recv()
        desc[("rs2", b, 0)].wait_recv()
        rows = seg(b, i)
        y = (
            x_ref[0, rows, :]
            + resid_ref[rows, :]
            + rs1r[b, srows(0), :].astype(jnp.float32)
            + rs2r[b, srows(0), :].astype(jnp.float32)
            + rs3r[b, :, :].astype(jnp.float32)
        )
        rms = jnp.sqrt(jnp.mean(y * y, axis=-1, keepdims=True) + 1e-6)
        out_ref[rows, :] = ((y / rms) * g).astype(jnp.bfloat16)
        ag(0, b, i, i ^ m3)
        ag(1, b, i, i ^ m2)
        ag(3, b, i, i ^ m1)

    for b in ORD_X_FIRST:
        m1, m2, m3 = MASKS[b]
        desc[("ag", b, 0)].wait_recv()
        ag(2, b, i ^ m3, i ^ m2)
        ag(4, b, i ^ m3, i ^ m1)

    for b in ORD_X_FIRST:
        m1, m2, m3 = MASKS[b]
        desc[("ag", b, 1)].wait_recv()
        desc[("ag", b, 2)].wait_recv()
        ag(5, b, i ^ m2, i ^ m1)
        ag(6, b, i ^ m2 ^ m3, i ^ m1)

    for b in ORD_YZ_FIRST:
        for idx in (3, 4, 5, 6):
            desc[("ag", b, idx)].wait_recv()
    for b in range(NB):
        for q in range(4):
            desc[("rs1", b, q)].wait_send()
        for q in range(2):
            desc[("rs2", b, q)].wait_send()
        desc[("rs3", b)].wait_send()
        for idx in range(7):
            desc[("ag", b, idx)].wait_send()


def kernel(partial, resid, gamma):
    g = gamma.reshape(1, D)
    return pl.pallas_call(
        _body,
        out_shape=jax.ShapeDtypeStruct((M, D), jnp.bfloat16),
        in_specs=[
            pl.BlockSpec(memory_space=pltpu.VMEM),
            pl.BlockSpec(memory_space=pltpu.VMEM),
            pl.BlockSpec(memory_space=pltpu.VMEM),
        ],
        out_specs=pl.BlockSpec(memory_space=pltpu.VMEM),
        scratch_shapes=[
            pltpu.VMEM((M, D), jnp.bfloat16),
            pltpu.VMEM((NB, 4 * SEG, D), jnp.bfloat16),
            pltpu.VMEM((NB, 2 * SEG, D), jnp.bfloat16),
            pltpu.VMEM((NB, 2 * SEG, D), jnp.bfloat16),
            pltpu.VMEM((NB, SEG, D), jnp.bfloat16),
            pltpu.VMEM((NB, SEG, D), jnp.bfloat16),
            pltpu.SemaphoreType.DMA((NB, 4)),
            pltpu.SemaphoreType.DMA((NB, 4)),
            pltpu.SemaphoreType.DMA((NB, 2)),
            pltpu.SemaphoreType.DMA((NB, 2)),
            pltpu.SemaphoreType.DMA((NB,)),
            pltpu.SemaphoreType.DMA((NB,)),
            pltpu.SemaphoreType.DMA((NB, 7)),
            pltpu.SemaphoreType.DMA((NB, 7)),
        ],
        compiler_params=pltpu.CompilerParams(
            collective_id=0, vmem_limit_bytes=96 * 1024 * 1024
        ),
    )(partial, resid, g)


# device time: 84661 ns/iter; 1.2783x vs baseline; 1.0085x over previous
import jax
import jax.numpy as jnp
from jax import lax
from jax.experimental import pallas as pl
from jax.experimental.pallas import tpu as pltpu

N_DEV = 8
M = 2048
D = 2048
NB = 4
BROWS = M // NB
SEG = BROWS // N_DEV

MASKS = (
    (1, 3, 4),
    (3, 4, 1),
    (4, 3, 1),
    (1, 4, 3),
)

ORD_X_FIRST = (0, 3, 1, 2)
ORD_YZ_FIRST = (1, 2, 0, 3)


def _body(
    x_ref,
    resid_ref,
    gamma_ref,
    out_ref,
    xb_ref,
    xr_ref,
    rs1r,
    rs2s,
    rs2r,
    rs3s,
    rs3r,
    rs1_ssem,
    rs1_rsem,
    rs2_ssem,
    rs2_rsem,
    rs3_ssem,
    rs3_rsem,
    ag_ssem,
    ag_rsem,
):
    i = lax.axis_index("i")

    xb_ref[:, :] = x_ref[0, :, :].astype(jnp.bfloat16)

    barrier_sem = pltpu.get_barrier_semaphore()
    for m in (1, 3, 4):
        pl.semaphore_signal(
            barrier_sem,
            inc=1,
            device_id=(i ^ m,),
            device_id_type=pl.DeviceIdType.MESH,
        )
    pl.semaphore_wait(barrier_sem, 3)

    def seg(b, o):
        return pl.ds(b * BROWS + o * SEG, SEG)

    def srows(q):
        return pl.ds(q * SEG, SEG)

    desc = {}

    def start(key, src, dst, ssem, rsem, partner):
        d = pltpu.make_async_remote_copy(
            src_ref=src,
            dst_ref=dst,
            send_sem=ssem,
            recv_sem=rsem,
            device_id=(partner,),
            device_id_type=pl.DeviceIdType.MESH,
        )
        desc[key] = d
        d.start()

    E1Q = {b: (0, MASKS[b][1], MASKS[b][2], MASKS[b][1] ^ MASKS[b][2]) for b in range(NB)}
    for q in (3, 1, 2, 0):
        for b in ORD_X_FIRST:
            m1 = MASKS[b][0]
            p1 = i ^ m1
            start(
                ("rs1", b, q),
                xb_ref.at[seg(b, p1 ^ E1Q[b][q])],
                rs1r.at[b, srows(q)],
                rs1_ssem.at[b, q],
                rs1_rsem.at[b, q],
                p1,
            )

    for b in range(NB):
        rows = seg(b, i)
        xr_ref[b, :, :] = x_ref[0, rows, :] + resid_ref[rows, :]

    for b in (0, 3, 2, 1):
        m1, m2, m3 = MASKS[b]
        p2 = i ^ m2
        desc[("rs1", b, 3)].wait_recv()
        rs2s[b, srows(1), :] = xb_ref[seg(b, p2 ^ m3), :] + rs1r[b, srows(3), :]
        start(
            ("rs2", b, 1),
            rs2s.at[b, srows(1)],
            rs2r.at[b, srows(1)],
            rs2_ssem.at[b, 1],
            rs2_rsem.at[b, 1],
            p2,
        )
    for b in (0, 3, 2, 1):
        m1, m2, m3 = MASKS[b]
        p2 = i ^ m2
        desc[("rs1", b, 1)].wait_recv()
        rs2s[b, srows(0), :] = xb_ref[seg(b, p2), :] + rs1r[b, srows(1), :]
        start(
            ("rs2", b, 0),
            rs2s.at[b, srows(0)],
            rs2r.at[b, srows(0)],
            rs2_ssem.at[b, 0],
            rs2_rsem.at[b, 0],
            p2,
        )

    for b in (1, 2, 0, 3):
        m1, m2, m3 = MASKS[b]
        p3 = i ^ m3
        desc[("rs2", b, 1)].wait_recv()
        desc[("rs1", b, 2)].wait_recv()
        rs3s[b, :, :] = (
            xb_ref[seg(b, p3), :]
            + rs1r[b, srows(2), :]
            + rs2r[b, srows(1), :]
        )
        start(
            ("rs3", b),
            rs3s.at[b],
            rs3r.at[b],
            rs3_ssem.at[b],
            rs3_rsem.at[b],
            p3,
        )

    g = gamma_ref[:, :]

    def ag(idx, b, o, partner):
        start(
            ("ag", b, idx),
            out_ref.at[seg(b, o)],
            out_ref.at[seg(b, o)],
            ag_ssem.at[b, idx],
            ag_rsem.at[b, idx],
            partner,
        )

    for b in ORD_X_FIRST:
        m1, m2, m3 = MASKS[b]
        desc[("rs3", b)].wait_recv()
        desc[("rs1", b, 0)].wait_recv()
        desc[("rs2", b, 0)].wait_recv()
        rows = seg(b, i)
        y = (
            xr_ref[b, :, :]
            + rs1r[b, srows(0), :].astype(jnp.float32)
            + rs2r[b, srows(0), :].astype(jnp.float32)
            + rs3r[b, :, :].astype(jnp.float32)
        )
        rms = jnp.sqrt(jnp.mean(y * y, axis=-1, keepdims=True) + 1e-6)
        out_ref[rows, :] = ((y / rms) * g).astype(jnp.bfloat16)
        ag(0, b, i, i ^ m3)
        ag(1, b, i, i ^ m2)
        ag(3, b, i, i ^ m1)

    for b in ORD_X_FIRST:
        m1, m2, m3 = MASKS[b]
        desc[("ag", b, 0)].wait_recv()
        ag(2, b, i ^ m3, i ^ m2)
        ag(4, b, i ^ m3, i ^ m1)

    for b in ORD_X_FIRST:
        m1, m2, m3 = MASKS[b]
        desc[("ag", b, 1)].wait_recv()
        desc[("ag", b, 2)].wait_recv()
        ag(5, b, i ^ m2, i ^ m1)
        ag(6, b, i ^ m2 ^ m3, i ^ m1)

    for b in ORD_YZ_FIRST:
        for idx in (3, 4, 5, 6):
            desc[("ag", b, idx)].wait_recv()
    for b in range(NB):
        for q in range(4):
            desc[("rs1", b, q)].wait_send()
        for q in range(2):
            desc[("rs2", b, q)].wait_send()
        desc[("rs3", b)].wait_send()
        for idx in range(7):
            desc[("ag", b, idx)].wait_send()


def kernel(partial, resid, gamma):
    g = gamma.reshape(1, D)
    return pl.pallas_call(
        _body,
        out_shape=jax.ShapeDtypeStruct((M, D), jnp.bfloat16),
        in_specs=[
            pl.BlockSpec(memory_space=pltpu.VMEM),
            pl.BlockSpec(memory_space=pltpu.VMEM),
            pl.BlockSpec(memory_space=pltpu.VMEM),
        ],
        out_specs=pl.BlockSpec(memory_space=pltpu.VMEM),
        scratch_shapes=[
            pltpu.VMEM((M, D), jnp.bfloat16),
            pltpu.VMEM((NB, SEG, D), jnp.float32),
            pltpu.VMEM((NB, 4 * SEG, D), jnp.bfloat16),
            pltpu.VMEM((NB, 2 * SEG, D), jnp.bfloat16),
            pltpu.VMEM((NB, 2 * SEG, D), jnp.bfloat16),
            pltpu.VMEM((NB, SEG, D), jnp.bfloat16),
            pltpu.VMEM((NB, SEG, D), jnp.bfloat16),
            pltpu.SemaphoreType.DMA((NB, 4)),
            pltpu.SemaphoreType.DMA((NB, 4)),
            pltpu.SemaphoreType.DMA((NB, 2)),
            pltpu.SemaphoreType.DMA((NB, 2)),
            pltpu.SemaphoreType.DMA((NB,)),
            pltpu.SemaphoreType.DMA((NB,)),
            pltpu.SemaphoreType.DMA((NB, 7)),
            pltpu.SemaphoreType.DMA((NB, 7)),
        ],
        compiler_params=pltpu.CompilerParams(
            collective_id=0, vmem_limit_bytes=96 * 1024 * 1024
        ),
    )(partial, resid, g)
